# baseline (speedup 1.0000x reference)
import functools

import jax
import jax.numpy as jnp
from jax import lax
from jax.experimental import pallas as pl
from jax.experimental.pallas import tpu as pltpu
from jax.experimental.pallas import tpu_sc as plsc

_CHUNK = 512
_NBUF = 2


def _transpose_table(tt):
    C, V = tt.shape
    BT = 2048

    def body(in_ref, out_ref):
        out_ref[...] = in_ref[...].T

    return pl.pallas_call(
        body,
        grid=(pl.cdiv(V, BT),),
        in_specs=[pl.BlockSpec((C, BT), lambda b: (0, b))],
        out_specs=pl.BlockSpec((BT, C), lambda b: (b, 0)),
        out_shape=jax.ShapeDtypeStruct((V, C), jnp.float32),
    )(tt)


def _transpose_out(g, J, I):
    C = g.shape[1]
    BI = 512
    assert I % BI == 0

    def body(in_ref, out_ref):
        out_ref[0] = in_ref[0].T

    return pl.pallas_call(
        body,
        grid=(J, I // BI),
        in_specs=[pl.BlockSpec((1, BI, C), lambda j, b: (j, b, 0))],
        out_specs=pl.BlockSpec((1, C, BI), lambda j, b: (j, 0, b)),
        out_shape=jax.ShapeDtypeStruct((J, C, I), jnp.float32),
    )(g.reshape(J, I, C))


def _sc_gather(table, idx_flat):
    V, D = table.shape
    B = idx_flat.shape[0]
    info = plsc.get_sparse_core_info()
    NC, NS = info.num_cores, info.num_subcores
    NW = NC * NS
    b_per_w = B // NW
    n_chunks = b_per_w // _CHUNK
    n_groups = n_chunks // _NBUF
    assert b_per_w * NW == B and n_chunks * _CHUNK == b_per_w
    assert n_groups * _NBUF == n_chunks and n_groups >= 2

    mesh = plsc.VectorSubcoreMesh(core_axis_name="c", subcore_axis_name="s")

    @functools.partial(
        pl.kernel,
        mesh=mesh,
        out_type=jax.ShapeDtypeStruct((B, D), jnp.float32),
        scratch_types=(
            [pltpu.VMEM((b_per_w,), jnp.int32),
             pltpu.VMEM((_NBUF, _CHUNK, D), jnp.float32)]
            + [pltpu.SemaphoreType.DMA] * (2 * _NBUF)
        ),
        compiler_params=pltpu.CompilerParams(use_tc_tiling_on_sc=False),
    )
    def k(table_hbm, idx_hbm, out_hbm, idx_v, rows_v, *sems):
        gsem, wsem = sems[:_NBUF], sems[_NBUF:]
        wid = lax.axis_index("s") * NC + lax.axis_index("c")
        base = wid * b_per_w
        pltpu.sync_copy(idx_hbm.at[pl.ds(base, b_per_w)], idx_v)

        def gather_desc(j, b):
            return pltpu.make_async_copy(
                table_hbm.at[idx_v.at[pl.ds(j * _CHUNK, _CHUNK)]],
                rows_v.at[b], gsem[b])

        def write_desc(j, b):
            return pltpu.make_async_copy(
                rows_v.at[b], out_hbm.at[pl.ds(base + j * _CHUNK, _CHUNK)],
                wsem[b])

        for b in range(_NBUF):
            gather_desc(b, b).start()

        def body(g, carry):
            j0 = g * _NBUF
            for b in range(_NBUF):
                gather_desc(j0 + b, b).wait()
                write_desc(j0 + b, b).start()
            for b in range(_NBUF):
                write_desc(j0 + b, b).wait()
                gather_desc(j0 + _NBUF + b, b).start()
            return carry

        lax.fori_loop(0, n_groups - 1, body, 0, unroll=False)

        jf = (n_groups - 1) * _NBUF
        for b in range(_NBUF):
            gather_desc(jf + b, b).wait()
            write_desc(jf + b, b).start()
        for b in range(_NBUF):
            write_desc(jf + b, b).wait()

    return k(table, idx_flat)


@jax.jit
def _run(input_, indices):
    V, D = input_.shape
    I, J = indices.shape
    table = _transpose_table(input_.T)
    idx_flat = indices.T.reshape(I * J)
    g = _sc_gather(table, idx_flat)
    out_t = _transpose_out(g, J, I)
    return out_t.transpose(2, 0, 1)


def kernel(input_, indices):
    return _run(input_, indices)
